# Initial kernel scaffold; baseline (speedup 1.0000x reference)
import jax
import jax.numpy as jnp
from jax.experimental import pallas as pl


def kernel(x):
    raise NotImplementedError("write your pallas kernel here")


# speedup vs baseline: 2.3858x; 2.3858x over previous
import jax
import jax.numpy as jnp
from jax.experimental import pallas as pl

_ROWS = 128
_COLS = 32768
_B = 128
_NB = _COLS // _B
_R = 8


def _scan_kernel(x_ref, u_ref, su_ref, o_ref):
    xb = x_ref[...]
    x2 = xb.reshape(_R * _NB, _B)
    y = jax.lax.dot_general(
        x2, u_ref[...], (((1,), (0,)), ((), ())),
        preferred_element_type=jnp.float32,
        precision=jax.lax.Precision.HIGHEST)
    s = jnp.sum(xb, axis=2)
    c = jax.lax.dot_general(
        s, su_ref[...], (((1,), (0,)), ((), ())),
        preferred_element_type=jnp.float32,
        precision=jax.lax.Precision.HIGHEST)
    o_ref[...] = y.reshape(_R, _NB, _B) + c[:, :, None]


def kernel(x):
    x3 = x.reshape(_ROWS, _NB, _B)
    u = jnp.triu(jnp.ones((_B, _B), jnp.float32))
    su = jnp.triu(jnp.ones((_NB, _NB), jnp.float32), k=1)
    out = pl.pallas_call(
        _scan_kernel,
        grid=(_ROWS // _R,),
        in_specs=[
            pl.BlockSpec((_R, _NB, _B), lambda i: (i, 0, 0)),
            pl.BlockSpec((_B, _B), lambda i: (0, 0)),
            pl.BlockSpec((_NB, _NB), lambda i: (0, 0)),
        ],
        out_specs=pl.BlockSpec((_R, _NB, _B), lambda i: (i, 0, 0)),
        out_shape=jax.ShapeDtypeStruct((_ROWS, _NB, _B), jnp.float32),
    )(x3, u, su)
    return out.reshape(_ROWS, _COLS)


# speedup vs baseline: 2.5207x; 1.0565x over previous
import jax
import jax.numpy as jnp
from jax.experimental import pallas as pl

_ROWS = 128
_COLS = 32768
_B = 128
_NB = _COLS // _B
_R = 8


def _scan_kernel(x_ref, u_ref, su_ref, o_ref):
    xb = x_ref[...]
    x2 = xb.reshape(_R * _NB, _B)
    y = jax.lax.dot_general(
        x2, u_ref[...], (((1,), (0,)), ((), ())),
        preferred_element_type=jnp.float32,
        precision=jax.lax.Precision.DEFAULT)
    s = jnp.sum(xb, axis=2)
    c = jax.lax.dot_general(
        s, su_ref[...], (((1,), (0,)), ((), ())),
        preferred_element_type=jnp.float32,
        precision=jax.lax.Precision.HIGHEST)
    o_ref[...] = y.reshape(_R, _NB, _B) + c[:, :, None]


def kernel(x):
    x3 = x.reshape(_ROWS, _NB, _B)
    u = jnp.triu(jnp.ones((_B, _B), jnp.float32))
    su = jnp.triu(jnp.ones((_NB, _NB), jnp.float32), k=1)
    out = pl.pallas_call(
        _scan_kernel,
        grid=(_ROWS // _R,),
        in_specs=[
            pl.BlockSpec((_R, _NB, _B), lambda i: (i, 0, 0)),
            pl.BlockSpec((_B, _B), lambda i: (0, 0)),
            pl.BlockSpec((_NB, _NB), lambda i: (0, 0)),
        ],
        out_specs=pl.BlockSpec((_R, _NB, _B), lambda i: (i, 0, 0)),
        out_shape=jax.ShapeDtypeStruct((_ROWS, _NB, _B), jnp.float32),
    )(x3, u, su)
    return out.reshape(_ROWS, _COLS)


# speedup vs baseline: 5.2196x; 2.0707x over previous
import jax
import jax.numpy as jnp
from jax.experimental import pallas as pl
from jax.experimental.pallas import tpu as pltpu

_ROWS = 128
_COLS = 32768
_C = 1024
_NC = _COLS // _C


def _scan_kernel(x_ref, u_ref, o_ref, carry_ref):
    i = pl.program_id(0)

    @pl.when(i == 0)
    def _():
        carry_ref[...] = jnp.zeros_like(carry_ref)

    xc = x_ref[...]
    y = jax.lax.dot_general(
        xc, u_ref[...], (((1,), (0,)), ((), ())),
        preferred_element_type=jnp.float32,
        precision=jax.lax.Precision.DEFAULT)
    o_ref[...] = y + carry_ref[:, :1]
    carry_ref[:, :1] += jnp.sum(xc, axis=1, keepdims=True)


def kernel(x):
    u = jnp.triu(jnp.ones((_C, _C), jnp.float32))
    return pl.pallas_call(
        _scan_kernel,
        grid=(_NC,),
        in_specs=[
            pl.BlockSpec((_ROWS, _C), lambda i: (0, i)),
            pl.BlockSpec((_C, _C), lambda i: (0, 0)),
        ],
        out_specs=pl.BlockSpec((_ROWS, _C), lambda i: (0, i)),
        out_shape=jax.ShapeDtypeStruct((_ROWS, _COLS), jnp.float32),
        scratch_shapes=[pltpu.VMEM((_ROWS, 8), jnp.float32)],
    )(x, u)
